# Initial kernel scaffold; baseline (speedup 1.0000x reference)
import jax
import jax.numpy as jnp
from jax.experimental import pallas as pl


def kernel(feat, w, b):
    raise NotImplementedError("write your pallas kernel here")


# speedup vs baseline: 1.0062x; 1.0062x over previous
import jax
import jax.numpy as jnp
from jax.experimental import pallas as pl
from jax.experimental.pallas import tpu as pltpu

_EPS = 1e-5


def _pool_bn_dot_kernel(x_ref, w_ref, o_ref):
    ct = pl.program_id(1)

    x = x_ref[...]
    PP, B, _, _ = x.shape

    pooled = jnp.max(x, axis=2)

    inv_cnt = 1.0 / float(PP * B)
    mean = jnp.sum(pooled, axis=(0, 1), keepdims=True) * inv_cnt
    diff = pooled - mean
    var = jnp.sum(diff * diff, axis=(0, 1), keepdims=True) * inv_cnt
    nrm = diff * jax.lax.rsqrt(var + _EPS)

    acc = jnp.dot(nrm[0], w_ref[0], preferred_element_type=jnp.float32)
    for p in range(1, PP):
        acc += jnp.dot(nrm[p], w_ref[p], preferred_element_type=jnp.float32)

    @pl.when(ct == 0)
    def _():
        o_ref[...] = acc

    @pl.when(ct > 0)
    def _():
        o_ref[...] = o_ref[...] + acc


def kernel(feat, w, b):
    B, C, H, W = feat.shape
    P = 3
    PP = P * P
    kh, kw = H // P, W // P
    N = w.shape[1]
    n_pad = ((N + 127) // 128) * 128

    NUM_CORES = 2
    CT_PER_CORE = 1
    NB = NUM_CORES * CT_PER_CORE
    c_t = C // NB

    x_win = (feat.reshape(B, C, P, kh, P, kw)
                 .transpose(2, 4, 0, 3, 5, 1)
                 .reshape(PP, B, kh * kw, C))
    w_pad = jnp.pad(w, ((0, 0), (0, n_pad - N)))
    w_r = (w_pad.reshape(NB, c_t, PP, n_pad)
                .transpose(0, 2, 1, 3))

    parts = pl.pallas_call(
        _pool_bn_dot_kernel,
        out_shape=jax.ShapeDtypeStruct((NUM_CORES, B, n_pad), jnp.float32),
        grid=(NUM_CORES, CT_PER_CORE),
        in_specs=[
            pl.BlockSpec((PP, B, kh * kw, c_t),
                         lambda k, ct: (0, 0, 0, k * CT_PER_CORE + ct)),
            pl.BlockSpec((None, PP, c_t, n_pad),
                         lambda k, ct: (k * CT_PER_CORE + ct, 0, 0, 0)),
        ],
        out_specs=pl.BlockSpec((None, B, n_pad), lambda k, ct: (k, 0, 0)),
        compiler_params=pltpu.CompilerParams(
            dimension_semantics=("parallel", "arbitrary"),
        ),
    )(x_win, w_r)

    return parts.sum(axis=0)[:, :N] + b


# speedup vs baseline: 1.0842x; 1.0775x over previous
import jax
import jax.numpy as jnp
from jax.experimental import pallas as pl
from jax.experimental.pallas import tpu as pltpu

_EPS = 1e-5


def _pool_kernel(x_ref, o_ref):
    BT, C, HW = x_ref.shape
    pooled = []
    for bi in range(BT):
        X = x_ref[bi]
        T = jnp.transpose(X, (1, 0))
        W5 = T.reshape(3, 8, 3, 8, C)
        m1 = jnp.max(W5, axis=1)
        m2 = jnp.max(m1, axis=2)
        pooled.append(m2)
    o_ref[...] = jnp.stack(pooled, axis=2)


def _bn_linear_kernel(p_ref, w_ref, b_ref, o_ref):
    PP, C, _ = w_ref.shape
    _, _, B, _ = p_ref.shape
    x = p_ref[...].reshape(PP, B, C)

    inv_cnt = 1.0 / float(PP * B)
    mean = jnp.sum(x, axis=(0, 1), keepdims=True) * inv_cnt
    diff = x - mean
    var = jnp.sum(diff * diff, axis=(0, 1), keepdims=True) * inv_cnt
    nrm = diff * jax.lax.rsqrt(var + _EPS)

    acc = jnp.dot(nrm[0], w_ref[0], preferred_element_type=jnp.float32)
    for p in range(1, PP):
        acc += jnp.dot(nrm[p], w_ref[p], preferred_element_type=jnp.float32)
    o_ref[...] = acc + b_ref[...]


def kernel(feat, w, b):
    B, C, H, W = feat.shape
    P = 3
    PP = P * P
    HW = H * W
    N = w.shape[1]
    n_pad = ((N + 127) // 128) * 128

    x2 = feat.reshape(B, C, HW)

    NUM_CORES = 2
    BT = 8
    STEPS = B // (NUM_CORES * BT)

    pooled = pl.pallas_call(
        _pool_kernel,
        out_shape=jax.ShapeDtypeStruct((P, P, B, C), jnp.float32),
        grid=(NUM_CORES, STEPS),
        in_specs=[
            pl.BlockSpec((BT, C, HW), lambda k, i: (k * STEPS + i, 0, 0)),
        ],
        out_specs=pl.BlockSpec((P, P, BT, C), lambda k, i: (0, 0, k * STEPS + i, 0)),
        compiler_params=pltpu.CompilerParams(
            dimension_semantics=("parallel", "arbitrary"),
        ),
    )(x2)

    w_pad = jnp.pad(w, ((0, 0), (0, n_pad - N)))
    w_r = w_pad.reshape(C, PP, n_pad).transpose(1, 0, 2)
    b_pad = jnp.pad(b, (0, n_pad - N)).reshape(1, n_pad)

    tn = n_pad // NUM_CORES
    out = pl.pallas_call(
        _bn_linear_kernel,
        out_shape=jax.ShapeDtypeStruct((B, n_pad), jnp.float32),
        grid=(NUM_CORES,),
        in_specs=[
            pl.BlockSpec((P, P, B, C), lambda k: (0, 0, 0, 0)),
            pl.BlockSpec((PP, C, tn), lambda k: (0, 0, k)),
            pl.BlockSpec((1, tn), lambda k: (0, k)),
        ],
        out_specs=pl.BlockSpec((B, tn), lambda k: (0, k)),
        compiler_params=pltpu.CompilerParams(
            dimension_semantics=("parallel",),
        ),
    )(pooled, w_r, b_pad)

    return out[:, :N]
